# Initial kernel scaffold; baseline (speedup 1.0000x reference)
import jax
import jax.numpy as jnp
from jax.experimental import pallas as pl


def kernel(z, weight):
    raise NotImplementedError("write your pallas kernel here")


# speedup vs baseline: 1.0102x; 1.0102x over previous
import functools

import jax
import jax.numpy as jnp
from jax import lax
from jax.experimental import pallas as pl
from jax.experimental.pallas import tpu as pltpu
from jax.experimental.pallas import tpu_sc as plsc

NUM_CODES = 8192
EMBED_DIM = 256
BETA = 0.25
TOKENS = 8192

TM = 1024
TN = 2048
T_TILES = TOKENS // TM
C_TILES = NUM_CODES // TN


def _argmin_body(z_ref, w_ref, idx_ref, minv_s, mini_s):
    c = pl.program_id(1)
    z = z_ref[...]
    w = w_ref[...]
    z2 = jnp.sum(z * z, axis=1, keepdims=True)
    w2 = jnp.sum(w * w, axis=1, keepdims=True).T
    mm = lax.dot_general(
        z.astype(jnp.bfloat16), w.astype(jnp.bfloat16),
        (((1,), (1,)), ((), ())),
        preferred_element_type=jnp.float32)
    d = (z2 + w2) - 2.0 * mm
    bmin = jnp.min(d, axis=1, keepdims=True)
    jio = lax.broadcasted_iota(jnp.int32, (TM, TN), 1)
    bidx = jnp.min(jnp.where(d == bmin, jio, TN), axis=1,
                   keepdims=True) + c * TN

    @pl.when(c == 0)
    def _():
        minv_s[...] = bmin
        mini_s[...] = bidx

    @pl.when(c > 0)
    def _():
        better = bmin < minv_s[...]
        minv_s[...] = jnp.where(better, bmin, minv_s[...])
        mini_s[...] = jnp.where(better, bidx, mini_s[...])

    @pl.when(c == C_TILES - 1)
    def _():
        idx_ref[...] = mini_s[...]


def _argmin_call(z_flat, weight):
    return pl.pallas_call(
        _argmin_body,
        grid=(T_TILES, C_TILES),
        in_specs=[
            pl.BlockSpec((TM, EMBED_DIM), lambda t, c: (t, 0)),
            pl.BlockSpec((TN, EMBED_DIM), lambda t, c: (c, 0)),
        ],
        out_specs=pl.BlockSpec((TM, 1), lambda t, c: (t, 0)),
        out_shape=jax.ShapeDtypeStruct((TOKENS, 1), jnp.int32),
        scratch_shapes=[
            pltpu.VMEM((TM, 1), jnp.float32),
            pltpu.VMEM((TM, 1), jnp.int32),
        ],
        compiler_params=pltpu.CompilerParams(
            dimension_semantics=("parallel", "arbitrary")),
    )(z_flat, weight)


def _gather_call(weight, idx_flat):
    info = plsc.get_sparse_core_info()
    nw = info.num_cores * info.num_subcores
    b_per_w = TOKENS // nw
    mesh = plsc.VectorSubcoreMesh(core_axis_name="c", subcore_axis_name="s")

    @functools.partial(
        pl.kernel, mesh=mesh,
        out_type=jax.ShapeDtypeStruct((TOKENS, EMBED_DIM), jnp.float32),
        scratch_types=[
            pltpu.VMEM((b_per_w,), jnp.int32),
            pltpu.VMEM((b_per_w, EMBED_DIM), jnp.float32),
            pltpu.SemaphoreType.DMA,
        ],
    )
    def k(table_hbm, idx_hbm, out_hbm, idx_v, rows_v, sem):
        wid = lax.axis_index("s") * info.num_cores + lax.axis_index("c")
        base = wid * b_per_w
        pltpu.sync_copy(idx_hbm.at[pl.ds(base, b_per_w)], idx_v)
        pltpu.async_copy(table_hbm.at[idx_v], rows_v, sem).wait()
        pltpu.sync_copy(rows_v, out_hbm.at[pl.ds(base, b_per_w)])

    return k(weight, idx_flat)


TM2 = 1024
TN2 = 2048
T2 = TOKENS // TM2
C2 = NUM_CODES // TN2


def _onehot_body(idx_ref, enc_ref, cnt_ref):
    t = pl.program_id(1)
    c = pl.program_id(0)
    jc = lax.broadcasted_iota(jnp.int32, (TM2, TN2), 1) + c * TN2
    oh = jnp.where(idx_ref[...] == jc, 1.0, 0.0).astype(jnp.float32)
    enc_ref[...] = oh
    colsum = jnp.sum(oh, axis=0, keepdims=True)

    @pl.when(t == 0)
    def _():
        cnt_ref[...] = colsum

    @pl.when(t > 0)
    def _():
        cnt_ref[...] = cnt_ref[...] + colsum


def _onehot_call(idx2d):
    return pl.pallas_call(
        _onehot_body,
        grid=(C2, T2),
        in_specs=[pl.BlockSpec((TM2, 1), lambda c, t: (t, 0))],
        out_specs=[
            pl.BlockSpec((TM2, TN2), lambda c, t: (t, c)),
            pl.BlockSpec((1, TN2), lambda c, t: (0, c)),
        ],
        out_shape=[
            jax.ShapeDtypeStruct((TOKENS, NUM_CODES), jnp.float32),
            jax.ShapeDtypeStruct((1, NUM_CODES), jnp.float32),
        ],
        compiler_params=pltpu.CompilerParams(
            dimension_semantics=("parallel", "arbitrary")),
    )(idx2d)


TM3 = 1024
T3 = TOKENS // TM3


def _scalar_body(z_ref, zq_ref, cnt_ref, loss_ref, perp_ref, acc_s):
    t = pl.program_id(0)
    diff = zq_ref[...] - z_ref[...]
    s = jnp.sum(diff * diff).reshape(1, 1)

    @pl.when(t == 0)
    def _():
        acc_s[...] = s

    @pl.when(t > 0)
    def _():
        acc_s[...] = acc_s[...] + s

    @pl.when(t == T3 - 1)
    def _():
        loss_ref[...] = BETA * acc_s[...] / (TOKENS * EMBED_DIM)
        p = cnt_ref[...] * (1.0 / TOKENS)
        ent = jnp.sum(p * jnp.log(p + 1e-10)).reshape(1, 1)
        perp_ref[...] = jnp.exp(-ent)


def _scalar_call(z_flat, z_q, counts):
    return pl.pallas_call(
        _scalar_body,
        grid=(T3,),
        in_specs=[
            pl.BlockSpec((TM3, EMBED_DIM), lambda t: (t, 0)),
            pl.BlockSpec((TM3, EMBED_DIM), lambda t: (t, 0)),
            pl.BlockSpec((1, NUM_CODES), lambda t: (0, 0)),
        ],
        out_specs=[
            pl.BlockSpec((1, 1), lambda t: (0, 0)),
            pl.BlockSpec((1, 1), lambda t: (0, 0)),
        ],
        out_shape=[
            jax.ShapeDtypeStruct((1, 1), jnp.float32),
            jax.ShapeDtypeStruct((1, 1), jnp.float32),
        ],
        scratch_shapes=[pltpu.VMEM((1, 1), jnp.float32)],
        compiler_params=pltpu.CompilerParams(
            dimension_semantics=("arbitrary",)),
    )(z_flat, z_q, counts)


def kernel(z, weight):
    z_m = jnp.moveaxis(z, 1, -1)
    z_flat = z_m.reshape(-1, EMBED_DIM)

    idx2d = _argmin_call(z_flat, weight)
    idx = idx2d.reshape(TOKENS)

    z_q = _gather_call(weight, idx)
    encodings, counts = _onehot_call(idx2d)
    loss11, perp11 = _scalar_call(z_flat, z_q, counts)

    z_q_out = jnp.moveaxis(z_q.reshape(z_m.shape), -1, 1)
    loss = loss11.reshape(())
    perplexity = perp11.reshape(())
    return (z_q_out, loss, perplexity, encodings, idx)
